# baseline (speedup 1.0000x reference)
import functools

import jax
import jax.numpy as jnp
from jax import lax
from jax.experimental import pallas as pl
from jax.experimental.pallas import tpu as pltpu
from jax.experimental.pallas import tpu_sc as plsc

N = 10000
E = 320000
D = 128
NC = 2
NS = 16
NW = NC * NS
RPS = 640
NP = NS * RPS
SB = 100
DW = 128
TEMP = 0.85

@functools.lru_cache(maxsize=None)
def _mesh():
    return plsc.VectorSubcoreMesh(
        core_axis_name="c", subcore_axis_name="s",
        num_cores=NC, num_subcores=NS)


@functools.lru_cache(maxsize=None)
def _get_deg_kernel():
    @functools.partial(
        pl.kernel,
        out_type=jax.ShapeDtypeStruct((NC * NP, DW), jnp.float32),
        mesh=_mesh(),
        scratch_types=[
            pltpu.VMEM((20, SB), jnp.int32),
            pltpu.VMEM((SB, DW), jnp.float32),
            pltpu.SemaphoreType.DMA,
            pltpu.VMEM_SHARED((NP, DW), jnp.float32),
        ],
    )
    def _deg_kernel(dst_hbm, ones_hbm, zeros_hbm, out_hbm, dst_v, ones_v,
                    sem, acc):
        c = lax.axis_index("c")
        s = lax.axis_index("s")
        pltpu.sync_copy(ones_hbm, ones_v)
        pltpu.sync_copy(zeros_hbm, acc.at[pl.ds(s * RPS, RPS)])
        plsc.subcore_barrier()

        def outer(o, carry):
            pltpu.sync_copy(dst_hbm.at[c, s, o], dst_v)
            descs = [pltpu.async_copy(ones_v, acc.at[dst_v.at[i]], sem,
                                      add=True) for i in range(20)]
            for d in descs:
                d.wait()
            return carry

        lax.fori_loop(0, 5, outer, 0)
        plsc.subcore_barrier()
        pltpu.sync_copy(acc.at[pl.ds(s * RPS, RPS)],
                        out_hbm.at[pl.ds(c * NP + s * RPS, RPS)])

    return _deg_kernel


@functools.lru_cache(maxsize=None)
def _make_segsum(no, kb):
    @functools.partial(
        pl.kernel,
        out_type=jax.ShapeDtypeStruct((NC * NP, D), jnp.float32),
        mesh=_mesh(),
        scratch_types=[
            pltpu.VMEM((2, kb, SB), jnp.int32),
            pltpu.VMEM((2, kb, SB), jnp.int32),
            pltpu.VMEM((3, SB, D), jnp.float32),
            pltpu.SemaphoreType.DMA,
            pltpu.SemaphoreType.DMA,
            pltpu.SemaphoreType.DMA,
            pltpu.VMEM_SHARED((NP, D), jnp.float32),
        ],
    )
    def segsum(table_hbm, src_hbm, dst_hbm, zeros_hbm, dummy_hbm, out_hbm,
               src_v, dst_v, rows_v, sem_g, sem_s, sem_i, acc):
        c = lax.axis_index("c")
        s = lax.axis_index("s")

        def stage(o, p):
            pltpu.async_copy(src_hbm.at[c, s, o], src_v.at[p], sem_i)
            pltpu.async_copy(dst_hbm.at[c, s, o], dst_v.at[p], sem_i)

        def wait_stage(p):
            pltpu.make_async_copy(
                src_hbm.at[c, s, 0], src_v.at[p], sem_i).wait()
            pltpu.make_async_copy(
                dst_hbm.at[c, s, 0], dst_v.at[p], sem_i).wait()

        def wait_gather(b):
            pltpu.make_async_copy(dummy_hbm, rows_v.at[b], sem_g).wait()

        def wait_scatter(b):
            pltpu.make_async_copy(dummy_hbm, rows_v.at[b], sem_s).wait()

        stage(0, 0)
        pltpu.sync_copy(zeros_hbm, acc.at[pl.ds(s * RPS, RPS)])
        plsc.subcore_barrier()

        def outer(o, carry):
            p = lax.rem(o, 2)
            wait_stage(p)
            stage(jnp.minimum(o + 1, no - 1), 1 - p)
            pltpu.async_copy(table_hbm.at[src_v.at[p, 0]], rows_v.at[0],
                             sem_g)
            for i in range(kb):
                b = i % 3
                wait_gather(b)
                if i >= 2:
                    wait_scatter((i + 1) % 3)
                if i + 1 < kb:
                    pltpu.async_copy(table_hbm.at[src_v.at[p, i + 1]],
                                     rows_v.at[(i + 1) % 3], sem_g)
                pltpu.async_copy(rows_v.at[b], acc.at[dst_v.at[p, i]], sem_s,
                                 add=True)
            wait_scatter((kb - 2) % 3)
            wait_scatter((kb - 1) % 3)
            return carry

        lax.fori_loop(0, no, outer, 0)
        wait_stage(no % 2)
        plsc.subcore_barrier()
        pltpu.sync_copy(acc.at[pl.ds(s * RPS, RPS)],
                        out_hbm.at[pl.ds(c * NP + s * RPS, RPS)])

    return segsum


_BLK = 1000
_GRID = N // _BLK


def _rowspec(w):
    return pl.BlockSpec((_BLK, w), lambda i: (i, 0))


def _fullspec(shape):
    return pl.BlockSpec(shape, lambda i: (0,) * len(shape))


def _deg_inv(da_ref, db_ref):
    deg = jnp.maximum(da_ref[:, 0:1] + db_ref[:, 0:1], 1.0)
    return deg, lax.rsqrt(deg)


def _prep_body(x_ref, da_ref, db_ref, out_ref):
    _, inv = _deg_inv(da_ref, db_ref)
    out_ref[0] = x_ref[...]
    out_ref[1] = x_ref[...] * inv


_prep = pl.pallas_call(
    _prep_body,
    grid=(_GRID,),
    in_specs=[_rowspec(D), _rowspec(DW), _rowspec(DW)],
    out_specs=pl.BlockSpec((2, _BLK, D), lambda i: (0, i, 0)),
    out_shape=jax.ShapeDtypeStruct((2, N, D), jnp.float32),
)


def _mid_body(p_ref, da_ref, db_ref, out_ref):
    deg, _ = _deg_inv(da_ref, db_ref)
    out_ref[...] = p_ref[...] / deg


_mid = pl.pallas_call(
    _mid_body,
    grid=(_GRID,),
    in_specs=[_rowspec(D), _rowspec(DW), _rowspec(DW)],
    out_specs=_rowspec(D),
    out_shape=jax.ShapeDtypeStruct((N, D), jnp.float32),
)


def _dense_body(last, h_ref, s_ref, p1p_ref, b0_ref, b1_ref, da_ref, db_ref,
                wgcr, wgcn, w10, w11, w12, wss, wsn, w20, w21, w22,
                wr1, wr2, wrg, wwk, wcf, out_ref):
    h = h_ref[...]
    s = s_ref[...]
    deg, inv = _deg_inv(da_ref, db_ref)
    p1 = p1p_ref[...] * inv
    p2 = (b0_ref[...] + b1_ref[...]) * inv
    sm = s / deg

    def dot(a, w):
        return jnp.dot(a, w[...], preferred_element_type=jnp.float32)

    e_gc = dot(h, wgcr) + dot(s, wgcn)
    e_t1 = dot(h, w10) + dot(p1, w11) + dot(p2, w12)
    e_sg = dot(h, wss) + dot(sm, wsn)
    e_t2 = dot(h, w20) + dot(p1, w21) + dot(p2, w22)

    logits = (0.5 / TEMP) * (dot(h, wr1) + dot(p1, wr2))
    g = dot(h, wrg) * (1.0 / TEMP)

    ga, gb = g[:, 0:1], g[:, 1:2]
    gm = jnp.maximum(ga, gb)
    ea = jnp.exp(ga - gm)
    eb = jnp.exp(gb - gm)
    pa = ea / (ea + eb)
    pb = eb / (ea + eb)

    l0, l1, l2, l3 = (logits[:, 0:1], logits[:, 1:2],
                      logits[:, 2:3], logits[:, 3:4])
    m01 = jnp.maximum(l0, l1)
    e0 = jnp.exp(l0 - m01)
    e1 = jnp.exp(l1 - m01)
    m23 = jnp.maximum(l2, l3)
    e2 = jnp.exp(l2 - m23)
    e3 = jnp.exp(l3 - m23)
    s01 = e0 + e1
    s23 = e2 + e3

    moe = ((pa / s01) * (e0 * e_gc + e1 * e_t1)
           + (pb / s23) * (e2 * e_sg + e3 * e_t2))

    weak = dot(h, wwk)
    conf = 1.0 / (1.0 + jnp.exp(-dot(h, wcf)))
    out = conf * moe + (1.0 - conf) * weak
    if last:
        out_ref[...] = out
    else:
        out = jnp.maximum(out, 0.0)
        out_ref[0] = out
        out_ref[1] = out * inv


def _make_dense(last):
    wspecs = ([_fullspec((D, D))] * 10
              + [_fullspec((D, 4)), _fullspec((D, 4)), _fullspec((D, 2)),
                 _fullspec((D, D)), _fullspec((D, 1))])
    if last:
        out_specs = _rowspec(D)
        out_shape = jax.ShapeDtypeStruct((N, D), jnp.float32)
    else:
        out_specs = pl.BlockSpec((2, _BLK, D), lambda i: (0, i, 0))
        out_shape = jax.ShapeDtypeStruct((2, N, D), jnp.float32)
    return pl.pallas_call(
        functools.partial(_dense_body, last),
        grid=(_GRID,),
        in_specs=[_rowspec(D)] * 5 + [_rowspec(DW)] * 2 + wspecs,
        out_specs=out_specs,
        out_shape=out_shape,
    )


_dense_mid_layer = _make_dense(False)
_dense_last_layer = _make_dense(True)


def kernel(x, edge_index, params):
    src = edge_index[0].astype(jnp.int32)
    dst = edge_index[1].astype(jnp.int32)

    src_a0 = src.reshape(NS, 20, 10, SB)
    src_a = jnp.stack([src_a0, src_a0 + N])
    dst_a = jnp.broadcast_to(dst.reshape(1, NS, 20, 10, SB),
                             (NC, NS, 20, 10, SB))
    src_b = src.reshape(NC, NS, 10, 10, SB)
    dst_b = dst.reshape(NC, NS, 10, 10, SB)
    dst_deg = dst.reshape(NC, NS, 5, 20, SB)

    zeros_d = jnp.zeros((RPS, D), jnp.float32)
    zeros_sb = jnp.zeros((SB, D), jnp.float32)
    zeros_w = jnp.zeros((RPS, DW), jnp.float32)
    ones_w = jnp.ones((SB, DW), jnp.float32)

    dcat = _get_deg_kernel()(dst_deg, ones_w, zeros_w)
    da, db = dcat[:N], dcat[NP:NP + N]

    n_layers = len(params)
    table = None
    h = x
    out = None
    for li, p in enumerate(params):
        last = li == n_layers - 1
        if li == 0:
            table = _prep(x, da, db)
        aout = _make_segsum(20, 10)(table.reshape(NC * N, D),
                                   src_a, dst_a, zeros_d, zeros_sb)
        s_sum, p1p = aout[:N], aout[NP:NP + N]
        g2 = _mid(p1p, da, db)
        bout = _make_segsum(10, 10)(g2, src_b, dst_b, zeros_d, zeros_sb)
        b0, b1 = bout[:N], bout[NP:NP + N]
        w = [p['gc_root'], p['gc_nbr'], p['tag1'][0], p['tag1'][1],
             p['tag1'][2], p['sage_self'], p['sage_nbr'], p['tag2'][0],
             p['tag2'][1], p['tag2'][2], p['r1'], p['r2'], p['rg'],
             p['weak'], p['conf']]
        if last:
            out = _dense_last_layer(h, s_sum, p1p, b0, b1, da, db, *w)
        else:
            table = _dense_mid_layer(h, s_sum, p1p, b0, b1, da, db, *w)
            h = table[0]
    return out


# speedup vs baseline: 1.1919x; 1.1919x over previous
import functools

import jax
import jax.numpy as jnp
from jax import lax
from jax.experimental import pallas as pl
from jax.experimental.pallas import tpu as pltpu
from jax.experimental.pallas import tpu_sc as plsc

N = 10000
E = 320000
D = 128
NC = 2
NS = 16
NW = NC * NS
RPS = 632
SB = 125
DW = 128
TEMP = 0.85

@functools.lru_cache(maxsize=None)
def _mesh():
    return plsc.VectorSubcoreMesh(
        core_axis_name="c", subcore_axis_name="s",
        num_cores=NC, num_subcores=NS)


LAST = N - (NS - 1) * RPS


def _owned(s):
    return s * RPS


def _copy_owned(s, src_fn, dst_fn):
    @pl.when(s < NS - 1)
    def _():
        pltpu.sync_copy(src_fn(RPS, _owned(s)), dst_fn(RPS, _owned(s)))

    @pl.when(s == NS - 1)
    def _():
        pltpu.sync_copy(src_fn(LAST, _owned(s)), dst_fn(LAST, _owned(s)))


@functools.lru_cache(maxsize=None)
def _get_deg_kernel():
    @functools.partial(
        pl.kernel,
        out_type=jax.ShapeDtypeStruct((NC * N, DW), jnp.float32),
        mesh=_mesh(),
        scratch_types=[
            pltpu.VMEM((16, 125), jnp.int32),
            pltpu.VMEM((125, DW), jnp.float32),
            pltpu.SemaphoreType.DMA,
            pltpu.VMEM_SHARED((N, DW), jnp.float32),
        ],
    )
    def _deg_kernel(dst_hbm, ones_hbm, zeros_hbm, out_hbm, dst_v, ones_v,
                    sem, acc):
        c = lax.axis_index("c")
        s = lax.axis_index("s")
        pltpu.sync_copy(ones_hbm, ones_v)
        _copy_owned(s, lambda n, o: zeros_hbm.at[pl.ds(0, n)],
                    lambda n, o: acc.at[pl.ds(o, n)])
        plsc.subcore_barrier()

        def outer(o, carry):
            pltpu.sync_copy(dst_hbm.at[c, s, o], dst_v)
            descs = [pltpu.async_copy(ones_v, acc.at[dst_v.at[i]], sem,
                                      add=True) for i in range(16)]
            for d in descs:
                d.wait()
            return carry

        lax.fori_loop(0, 5, outer, 0)
        plsc.subcore_barrier()
        _copy_owned(s, lambda n, o: acc.at[pl.ds(o, n)],
                    lambda n, o: out_hbm.at[pl.ds(c * N + o, n)])

    return _deg_kernel


@functools.lru_cache(maxsize=None)
def _make_segsum(no, kb, sb):
    @functools.partial(
        pl.kernel,
        out_type=jax.ShapeDtypeStruct((NC * N, D), jnp.float32),
        mesh=_mesh(),
        scratch_types=[
            pltpu.VMEM((2, kb, sb), jnp.int32),
            pltpu.VMEM((2, kb, sb), jnp.int32),
            pltpu.VMEM((2, sb, D), jnp.float32),
            pltpu.SemaphoreType.DMA,
            pltpu.SemaphoreType.DMA,
            pltpu.SemaphoreType.DMA,
            pltpu.VMEM_SHARED((N, D), jnp.float32),
        ],
    )
    def segsum(table_hbm, src_hbm, dst_hbm, zeros_hbm, dummy_hbm, out_hbm,
               src_v, dst_v, rows_v, sem_g, sem_s, sem_i, acc):
        c = lax.axis_index("c")
        s = lax.axis_index("s")

        def stage(o, p):
            pltpu.async_copy(src_hbm.at[c, s, o], src_v.at[p], sem_i)
            pltpu.async_copy(dst_hbm.at[c, s, o], dst_v.at[p], sem_i)

        def wait_stage(p):
            pltpu.make_async_copy(
                src_hbm.at[c, s, 0], src_v.at[p], sem_i).wait()
            pltpu.make_async_copy(
                dst_hbm.at[c, s, 0], dst_v.at[p], sem_i).wait()

        def wait_gather(b):
            pltpu.make_async_copy(dummy_hbm, rows_v.at[b], sem_g).wait()

        def wait_scatter(b):
            pltpu.make_async_copy(dummy_hbm, rows_v.at[b], sem_s).wait()

        stage(0, 0)
        _copy_owned(s, lambda n, o: zeros_hbm.at[pl.ds(0, n)],
                    lambda n, o: acc.at[pl.ds(o, n)])
        plsc.subcore_barrier()

        def outer(o, carry):
            p = lax.rem(o, 2)
            wait_stage(p)
            stage(jnp.minimum(o + 1, no - 1), 1 - p)
            pltpu.async_copy(table_hbm.at[src_v.at[p, 0]], rows_v.at[0],
                             sem_g)
            for i in range(kb):
                b = i % 2
                if i >= 1:
                    wait_scatter(1 - b)
                if i + 1 < kb:
                    pltpu.async_copy(table_hbm.at[src_v.at[p, i + 1]],
                                     rows_v.at[1 - b], sem_g)
                wait_gather(b)
                pltpu.async_copy(rows_v.at[b], acc.at[dst_v.at[p, i]], sem_s,
                                 add=True)
            wait_scatter((kb - 1) % 2)
            return carry

        lax.fori_loop(0, no, outer, 0)
        wait_stage(no % 2)
        plsc.subcore_barrier()
        _copy_owned(s, lambda n, o: acc.at[pl.ds(o, n)],
                    lambda n, o: out_hbm.at[pl.ds(c * N + o, n)])

    return segsum


_BLK = 1000
_GRID = N // _BLK


def _rowspec(w):
    return pl.BlockSpec((_BLK, w), lambda i: (i, 0))


def _fullspec(shape):
    return pl.BlockSpec(shape, lambda i: (0,) * len(shape))


def _deg_inv(da_ref, db_ref):
    deg = jnp.maximum(da_ref[:, 0:1] + db_ref[:, 0:1], 1.0)
    return deg, lax.rsqrt(deg)


def _prep_body(x_ref, da_ref, db_ref, out_ref):
    _, inv = _deg_inv(da_ref, db_ref)
    out_ref[0] = x_ref[...]
    out_ref[1] = x_ref[...] * inv


_prep = pl.pallas_call(
    _prep_body,
    grid=(_GRID,),
    in_specs=[_rowspec(D), _rowspec(DW), _rowspec(DW)],
    out_specs=pl.BlockSpec((2, _BLK, D), lambda i: (0, i, 0)),
    out_shape=jax.ShapeDtypeStruct((2, N, D), jnp.float32),
)


def _mid_body(p_ref, da_ref, db_ref, out_ref):
    deg, _ = _deg_inv(da_ref, db_ref)
    out_ref[...] = p_ref[...] / deg


_mid = pl.pallas_call(
    _mid_body,
    grid=(_GRID,),
    in_specs=[_rowspec(D), _rowspec(DW), _rowspec(DW)],
    out_specs=_rowspec(D),
    out_shape=jax.ShapeDtypeStruct((N, D), jnp.float32),
)


def _dense_body(last, h_ref, s_ref, p1p_ref, b0_ref, b1_ref, da_ref, db_ref,
                wgcr, wgcn, w10, w11, w12, wss, wsn, w20, w21, w22,
                wr1, wr2, wrg, wwk, wcf, out_ref):
    h = h_ref[...]
    s = s_ref[...]
    deg, inv = _deg_inv(da_ref, db_ref)
    p1 = p1p_ref[...] * inv
    p2 = (b0_ref[...] + b1_ref[...]) * inv
    sm = s / deg

    def dot(a, w):
        return jnp.dot(a, w[...], preferred_element_type=jnp.float32)

    e_gc = dot(h, wgcr) + dot(s, wgcn)
    e_t1 = dot(h, w10) + dot(p1, w11) + dot(p2, w12)
    e_sg = dot(h, wss) + dot(sm, wsn)
    e_t2 = dot(h, w20) + dot(p1, w21) + dot(p2, w22)

    logits = (0.5 / TEMP) * (dot(h, wr1) + dot(p1, wr2))
    g = dot(h, wrg) * (1.0 / TEMP)

    ga, gb = g[:, 0:1], g[:, 1:2]
    gm = jnp.maximum(ga, gb)
    ea = jnp.exp(ga - gm)
    eb = jnp.exp(gb - gm)
    pa = ea / (ea + eb)
    pb = eb / (ea + eb)

    l0, l1, l2, l3 = (logits[:, 0:1], logits[:, 1:2],
                      logits[:, 2:3], logits[:, 3:4])
    m01 = jnp.maximum(l0, l1)
    e0 = jnp.exp(l0 - m01)
    e1 = jnp.exp(l1 - m01)
    m23 = jnp.maximum(l2, l3)
    e2 = jnp.exp(l2 - m23)
    e3 = jnp.exp(l3 - m23)
    s01 = e0 + e1
    s23 = e2 + e3

    moe = ((pa / s01) * (e0 * e_gc + e1 * e_t1)
           + (pb / s23) * (e2 * e_sg + e3 * e_t2))

    weak = dot(h, wwk)
    conf = 1.0 / (1.0 + jnp.exp(-dot(h, wcf)))
    out = conf * moe + (1.0 - conf) * weak
    if last:
        out_ref[...] = out
    else:
        out = jnp.maximum(out, 0.0)
        out_ref[0] = out
        out_ref[1] = out * inv


def _make_dense(last):
    wspecs = ([_fullspec((D, D))] * 10
              + [_fullspec((D, 4)), _fullspec((D, 4)), _fullspec((D, 2)),
                 _fullspec((D, D)), _fullspec((D, 1))])
    if last:
        out_specs = _rowspec(D)
        out_shape = jax.ShapeDtypeStruct((N, D), jnp.float32)
    else:
        out_specs = pl.BlockSpec((2, _BLK, D), lambda i: (0, i, 0))
        out_shape = jax.ShapeDtypeStruct((2, N, D), jnp.float32)
    return pl.pallas_call(
        functools.partial(_dense_body, last),
        grid=(_GRID,),
        in_specs=[_rowspec(D)] * 5 + [_rowspec(DW)] * 2 + wspecs,
        out_specs=out_specs,
        out_shape=out_shape,
    )


_dense_mid_layer = _make_dense(False)
_dense_last_layer = _make_dense(True)


def kernel(x, edge_index, params):
    src = edge_index[0].astype(jnp.int32)
    dst = edge_index[1].astype(jnp.int32)

    src_a0 = src.reshape(NS, 20, 8, 125)
    src_a = jnp.stack([src_a0, src_a0 + N])
    dst_a = jnp.broadcast_to(dst.reshape(1, NS, 20, 8, 125),
                             (NC, NS, 20, 8, 125))
    src_b = src.reshape(NC, NS, 20, 4, 125)
    dst_b = dst.reshape(NC, NS, 20, 4, 125)
    dst_deg = dst.reshape(NC, NS, 5, 16, 125)

    zeros_d = jnp.zeros((RPS, D), jnp.float32)
    dummy_b = jnp.zeros((125, D), jnp.float32)
    ones_w = jnp.ones((125, DW), jnp.float32)

    dcat = _get_deg_kernel()(dst_deg, ones_w, zeros_d)
    da, db = dcat[:N], dcat[N:]

    n_layers = len(params)
    table = None
    h = x
    out = None
    for li, p in enumerate(params):
        last = li == n_layers - 1
        if li == 0:
            table = _prep(x, da, db)
        aout = _make_segsum(20, 8, 125)(table.reshape(NC * N, D),
                                        src_a, dst_a, zeros_d, dummy_b)
        s_sum, p1p = aout[:N], aout[N:]
        g2 = _mid(p1p, da, db)
        bout = _make_segsum(20, 4, 125)(g2, src_b, dst_b, zeros_d, dummy_b)
        b0, b1 = bout[:N], bout[N:]
        w = [p['gc_root'], p['gc_nbr'], p['tag1'][0], p['tag1'][1],
             p['tag1'][2], p['sage_self'], p['sage_nbr'], p['tag2'][0],
             p['tag2'][1], p['tag2'][2], p['r1'], p['r2'], p['rg'],
             p['weak'], p['conf']]
        if last:
            out = _dense_last_layer(h, s_sum, p1p, b0, b1, da, db, *w)
        else:
            table = _dense_mid_layer(h, s_sum, p1p, b0, b1, da, db, *w)
            h = table[0]
    return out


# speedup vs baseline: 1.2073x; 1.0129x over previous
import functools

import jax
import jax.numpy as jnp
from jax import lax
from jax.experimental import pallas as pl
from jax.experimental.pallas import tpu as pltpu
from jax.experimental.pallas import tpu_sc as plsc

N = 10000
E = 320000
D = 128
NC = 2
NS = 16
NW = NC * NS
RPS = 632
SB = 125
DW = 128
TEMP = 0.85

@functools.lru_cache(maxsize=None)
def _mesh():
    return plsc.VectorSubcoreMesh(
        core_axis_name="c", subcore_axis_name="s",
        num_cores=NC, num_subcores=NS)


LAST = N - (NS - 1) * RPS


def _owned(s):
    return s * RPS


def _copy_owned(s, src_fn, dst_fn):
    @pl.when(s < NS - 1)
    def _():
        pltpu.sync_copy(src_fn(RPS, _owned(s)), dst_fn(RPS, _owned(s)))

    @pl.when(s == NS - 1)
    def _():
        pltpu.sync_copy(src_fn(LAST, _owned(s)), dst_fn(LAST, _owned(s)))


@functools.lru_cache(maxsize=None)
def _get_deg_kernel():
    @functools.partial(
        pl.kernel,
        out_type=jax.ShapeDtypeStruct((NC * N, DW), jnp.float32),
        mesh=_mesh(),
        scratch_types=[
            pltpu.VMEM((16, 125), jnp.int32),
            pltpu.VMEM((125, DW), jnp.float32),
            pltpu.SemaphoreType.DMA,
            pltpu.VMEM_SHARED((N, DW), jnp.float32),
        ],
    )
    def _deg_kernel(dst_hbm, ones_hbm, zeros_hbm, out_hbm, dst_v, ones_v,
                    sem, acc):
        c = lax.axis_index("c")
        s = lax.axis_index("s")
        pltpu.sync_copy(ones_hbm, ones_v)
        _copy_owned(s, lambda n, o: zeros_hbm.at[pl.ds(0, n)],
                    lambda n, o: acc.at[pl.ds(o, n)])
        plsc.subcore_barrier()

        def outer(o, carry):
            pltpu.sync_copy(dst_hbm.at[c, s, o], dst_v)
            descs = [pltpu.async_copy(ones_v, acc.at[dst_v.at[i]], sem,
                                      add=True) for i in range(16)]
            for d in descs:
                d.wait()
            return carry

        lax.fori_loop(0, 5, outer, 0)
        plsc.subcore_barrier()
        _copy_owned(s, lambda n, o: acc.at[pl.ds(o, n)],
                    lambda n, o: out_hbm.at[pl.ds(c * N + o, n)])

    return _deg_kernel


@functools.lru_cache(maxsize=None)
def _make_segsum(no, kb, sb):
    @functools.partial(
        pl.kernel,
        out_type=jax.ShapeDtypeStruct((NC * N, D), jnp.float32),
        mesh=_mesh(),
        scratch_types=[
            pltpu.VMEM((2, kb, sb), jnp.int32),
            pltpu.VMEM((2, kb, sb), jnp.int32),
            pltpu.VMEM((2, sb, D), jnp.float32),
            pltpu.SemaphoreType.DMA,
            pltpu.SemaphoreType.DMA,
            pltpu.SemaphoreType.DMA,
            pltpu.VMEM_SHARED((N, D), jnp.float32),
        ],
    )
    def segsum(table_hbm, src_hbm, dst_hbm, zeros_hbm, dummy_hbm, out_hbm,
               src_v, dst_v, rows_v, sem_g, sem_s, sem_i, acc):
        c = lax.axis_index("c")
        s = lax.axis_index("s")

        def stage(o, p):
            pltpu.async_copy(src_hbm.at[c, s, o], src_v.at[p], sem_i)
            pltpu.async_copy(dst_hbm.at[c, s, o], dst_v.at[p], sem_i)

        def wait_stage(p):
            pltpu.make_async_copy(
                src_hbm.at[c, s, 0], src_v.at[p], sem_i).wait()
            pltpu.make_async_copy(
                dst_hbm.at[c, s, 0], dst_v.at[p], sem_i).wait()

        def wait_gather(b):
            pltpu.make_async_copy(dummy_hbm, rows_v.at[b], sem_g).wait()

        def wait_scatter(b):
            pltpu.make_async_copy(dummy_hbm, rows_v.at[b], sem_s).wait()

        stage(0, 0)
        _copy_owned(s, lambda n, o: zeros_hbm.at[pl.ds(0, n)],
                    lambda n, o: acc.at[pl.ds(o, n)])
        plsc.subcore_barrier()

        def outer(o, carry):
            p = lax.rem(o, 2)
            wait_stage(p)
            stage(jnp.minimum(o + 1, no - 1), 1 - p)
            pltpu.async_copy(table_hbm.at[src_v.at[p, 0]], rows_v.at[0],
                             sem_g)
            for i in range(kb):
                b = i % 2
                if i >= 1:
                    wait_scatter(1 - b)
                if i + 1 < kb:
                    pltpu.async_copy(table_hbm.at[src_v.at[p, i + 1]],
                                     rows_v.at[1 - b], sem_g)
                wait_gather(b)
                pltpu.async_copy(rows_v.at[b], acc.at[dst_v.at[p, i]], sem_s,
                                 add=True)
            wait_scatter((kb - 1) % 2)
            return carry

        lax.fori_loop(0, no, outer, 0)
        wait_stage(no % 2)
        plsc.subcore_barrier()
        _copy_owned(s, lambda n, o: acc.at[pl.ds(o, n)],
                    lambda n, o: out_hbm.at[pl.ds(c * N + o, n)])

    return segsum


_BLK = 1000
_GRID = N // _BLK


def _rowspec(w):
    return pl.BlockSpec((_BLK, w), lambda i: (i, 0))


def _fullspec(shape):
    return pl.BlockSpec(shape, lambda i: (0,) * len(shape))


def _deg_inv(da_ref, db_ref):
    deg = jnp.maximum(da_ref[:, 0:1] + db_ref[:, 0:1], 1.0)
    return deg, lax.rsqrt(deg)


def _prep_body(x_ref, da_ref, db_ref, out_ref):
    _, inv = _deg_inv(da_ref, db_ref)
    out_ref[0] = x_ref[...]
    out_ref[1] = x_ref[...] * inv


_prep = pl.pallas_call(
    _prep_body,
    grid=(_GRID,),
    in_specs=[_rowspec(D), _rowspec(DW), _rowspec(DW)],
    out_specs=pl.BlockSpec((2, _BLK, D), lambda i: (0, i, 0)),
    out_shape=jax.ShapeDtypeStruct((2, N, D), jnp.float32),
)


def _mid_body(p_ref, da_ref, db_ref, out_ref):
    deg, _ = _deg_inv(da_ref, db_ref)
    out_ref[...] = p_ref[...] / deg


_mid = pl.pallas_call(
    _mid_body,
    grid=(_GRID,),
    in_specs=[_rowspec(D), _rowspec(DW), _rowspec(DW)],
    out_specs=_rowspec(D),
    out_shape=jax.ShapeDtypeStruct((N, D), jnp.float32),
)


def _dense_body(last, h_ref, s_ref, p1p_ref, b0_ref, b1_ref, da_ref, db_ref,
                wgcr, wgcn, w10, w11, w12, wss, wsn, w20, w21, w22,
                wr1, wr2, wrg, wwk, wcf, out_ref):
    h = h_ref[...]
    s = s_ref[...]
    deg, inv = _deg_inv(da_ref, db_ref)
    p1 = p1p_ref[...] * inv
    p2 = (b0_ref[...] + b1_ref[...]) * inv
    sm = s / deg

    def dot(a, w):
        return jnp.dot(a, w[...], preferred_element_type=jnp.float32)

    e_gc = dot(h, wgcr) + dot(s, wgcn)
    e_t1 = dot(h, w10) + dot(p1, w11) + dot(p2, w12)
    e_sg = dot(h, wss) + dot(sm, wsn)
    e_t2 = dot(h, w20) + dot(p1, w21) + dot(p2, w22)

    logits = (0.5 / TEMP) * (dot(h, wr1) + dot(p1, wr2))
    g = dot(h, wrg) * (1.0 / TEMP)

    ga, gb = g[:, 0:1], g[:, 1:2]
    gm = jnp.maximum(ga, gb)
    ea = jnp.exp(ga - gm)
    eb = jnp.exp(gb - gm)
    pa = ea / (ea + eb)
    pb = eb / (ea + eb)

    l0, l1, l2, l3 = (logits[:, 0:1], logits[:, 1:2],
                      logits[:, 2:3], logits[:, 3:4])
    m01 = jnp.maximum(l0, l1)
    e0 = jnp.exp(l0 - m01)
    e1 = jnp.exp(l1 - m01)
    m23 = jnp.maximum(l2, l3)
    e2 = jnp.exp(l2 - m23)
    e3 = jnp.exp(l3 - m23)
    s01 = e0 + e1
    s23 = e2 + e3

    moe = ((pa / s01) * (e0 * e_gc + e1 * e_t1)
           + (pb / s23) * (e2 * e_sg + e3 * e_t2))

    weak = dot(h, wwk)
    conf = 1.0 / (1.0 + jnp.exp(-dot(h, wcf)))
    out = conf * moe + (1.0 - conf) * weak
    if last:
        out_ref[...] = out
    else:
        out = jnp.maximum(out, 0.0)
        out_ref[0] = out
        out_ref[1] = out * inv


def _make_dense(last):
    wspecs = ([_fullspec((D, D))] * 10
              + [_fullspec((D, 4)), _fullspec((D, 4)), _fullspec((D, 2)),
                 _fullspec((D, D)), _fullspec((D, 1))])
    if last:
        out_specs = _rowspec(D)
        out_shape = jax.ShapeDtypeStruct((N, D), jnp.float32)
    else:
        out_specs = pl.BlockSpec((2, _BLK, D), lambda i: (0, i, 0))
        out_shape = jax.ShapeDtypeStruct((2, N, D), jnp.float32)
    return pl.pallas_call(
        functools.partial(_dense_body, last),
        grid=(_GRID,),
        in_specs=[_rowspec(D)] * 5 + [_rowspec(DW)] * 2 + wspecs,
        out_specs=out_specs,
        out_shape=out_shape,
    )


_dense_mid_layer = _make_dense(False)
_dense_last_layer = _make_dense(True)


def kernel(x, edge_index, params):
    src = edge_index[0].astype(jnp.int32)
    dst = edge_index[1].astype(jnp.int32)

    src_a0 = src.reshape(NS, 20, 8, 125)
    src_a = jnp.stack([src_a0, src_a0 + N])
    dst_a = jnp.broadcast_to(dst.reshape(1, NS, 20, 8, 125),
                             (NC, NS, 20, 8, 125))
    src_b = src.reshape(NC, NS, 10, 8, 125)
    dst_b = dst.reshape(NC, NS, 10, 8, 125)
    dst_deg = dst.reshape(NC, NS, 5, 16, 125)

    zeros_d = jnp.zeros((RPS, D), jnp.float32)
    dummy_b = jnp.zeros((125, D), jnp.float32)
    ones_w = jnp.ones((125, DW), jnp.float32)

    dcat = _get_deg_kernel()(dst_deg, ones_w, zeros_d)
    da, db = dcat[:N], dcat[N:]

    n_layers = len(params)
    table = None
    h = x
    out = None
    for li, p in enumerate(params):
        last = li == n_layers - 1
        if li == 0:
            table = _prep(x, da, db)
        aout = _make_segsum(20, 8, 125)(table.reshape(NC * N, D),
                                        src_a, dst_a, zeros_d, dummy_b)
        s_sum, p1p = aout[:N], aout[N:]
        g2 = _mid(p1p, da, db)
        bout = _make_segsum(10, 8, 125)(g2, src_b, dst_b, zeros_d, dummy_b)
        b0, b1 = bout[:N], bout[N:]
        w = [p['gc_root'], p['gc_nbr'], p['tag1'][0], p['tag1'][1],
             p['tag1'][2], p['sage_self'], p['sage_nbr'], p['tag2'][0],
             p['tag2'][1], p['tag2'][2], p['r1'], p['r2'], p['rg'],
             p['weak'], p['conf']]
        if last:
            out = _dense_last_layer(h, s_sum, p1p, b0, b1, da, db, *w)
        else:
            table = _dense_mid_layer(h, s_sum, p1p, b0, b1, da, db, *w)
            h = table[0]
    return out
